# baseline (speedup 1.0000x reference)
import jax
import jax.numpy as jnp
from jax import lax
from jax.experimental import pallas as pl
from jax.experimental.pallas import tpu as pltpu
from jax.experimental.pallas import tpu_sc as plsc

BATCH = 16384
FEAT = 64
NUM_CLASSES = 100000

NC = 2
NS = 16
NW = NC * NS
BPW = BATCH // NW
H0 = 50048
H1 = NUM_CLASSES - H0
NPAIR = NUM_CLASSES // 2


def _sc_body(labels_hbm, featT_hbm, cpairT_hbm, tailT_hbm,
             parts_hbm, num_hbm,
             dimrow_v, partial_v, labs_v, feat_v, num_v,
             lab_sem, feat_sems, dim_sem):
    c = lax.axis_index("c")
    s = lax.axis_index("s")
    w = c * NS + s

    lab_cp = pltpu.async_copy(labels_hbm, labs_v, lab_sem)
    feat_cps = [
        pltpu.async_copy(featT_hbm.at[c * 32 + p * 16 + s],
                         feat_v[p], feat_sems[p])
        for p in range(2)
    ]
    lab_cp.wait()

    ones = jnp.ones((16,), jnp.float32)
    zeros = jnp.zeros((16,), jnp.float32)

    for h in range(2):
        lo = h * H0
        hsz = H0 if h == 0 else H1

        @plsc.parallel_loop(0, (hsz + 15) // 16, 1, unroll=8)
        def _zero(k):
            dimrow_v[pl.ds(k * 16, 16)] = zeros

        @plsc.parallel_loop(0, BATCH // 16, 1, unroll=8)
        def _count(st):
            lab = labs_v[pl.ds(st * 16, 16)]
            loc = lab - lo
            m = (loc >= 0) & (loc < hsz)
            plsc.addupdate_scatter(dimrow_v, [loc], ones, mask=m)

        @plsc.parallel_loop(0, BPW // 16, 1, unroll=8)
        def _mynum(j):
            lab = labs_v[pl.ds(w * BPW + j * 16, 16)]
            loc = lab - lo
            m = (loc >= 0) & (loc < hsz)
            g = plsc.load_gather(dimrow_v, [loc], mask=m)
            cur = num_v[pl.ds(j * 16, 16)]
            num_v[pl.ds(j * 16, 16)] = jnp.where(m, g, cur)

    ncp = pltpu.async_copy(num_v, num_hbm.at[pl.ds(w * BPW, BPW)], lab_sem)

    for p in range(2):
        feat_cps[p].wait()
        d = c * 32 + p * 16 + s
        lin = NPAIR - 80
        pltpu.sync_copy(cpairT_hbm.at[d, pl.ds(0, lin)],
                        dimrow_v.at[pl.ds(0, lin)])
        pltpu.sync_copy(tailT_hbm.at[d], dimrow_v.at[pl.ds(lin, 128)])

        @plsc.parallel_loop(0, BATCH // 16, 1, unroll=8)
        def _sweep(st):
            lab = labs_v[pl.ds(st * 16, 16)]
            wf = plsc.load_gather(dimrow_v, [lax.shift_right_logical(lab, 1)])
            w32 = plsc.bitcast(wf, jnp.int32)
            odd = lax.bitwise_and(lab, 1) == 1
            lo16 = lax.shift_left(w32, 16)
            hi16 = lax.bitwise_and(w32, jnp.int32(-65536))
            cv = plsc.bitcast(jnp.where(odd, hi16, lo16), jnp.float32)
            f = feat_v[p][pl.ds(st * 16, 16)]
            dlt = f - cv
            dd = dlt * dlt
            prow = st >> 6
            pcol = (st & 63) * 16
            if p == 0:
                partial_v[prow, pl.ds(pcol, 16)] = dd
            else:
                partial_v[prow, pl.ds(pcol, 16)] = (
                    partial_v[prow, pl.ds(pcol, 16)] + dd)

    ncp.wait()
    pltpu.sync_copy(partial_v, parts_hbm.at[pl.ds(w * 16, 16)])


@jax.jit
def _sc_stage(labels2d, featT, cpairT, tailT):
    mesh = plsc.VectorSubcoreMesh(core_axis_name="c", subcore_axis_name="s")
    fn = pl.kernel(
        _sc_body,
        out_type=(
            jax.ShapeDtypeStruct((512, 1024), jnp.float32),
            jax.ShapeDtypeStruct((BATCH,), jnp.float32),
        ),
        mesh=mesh,
        compiler_params=pltpu.CompilerParams(
            needs_layout_passes=False, use_tc_tiling_on_sc=True),
        scratch_types=[
            pltpu.VMEM((H0,), jnp.float32),
            pltpu.VMEM((16, 1024), jnp.float32),
            pltpu.VMEM((BATCH,), jnp.int32),
            [pltpu.VMEM((BATCH,), jnp.float32) for _ in range(2)],
            pltpu.VMEM((BPW,), jnp.float32),
            pltpu.SemaphoreType.DMA,
            [pltpu.SemaphoreType.DMA for _ in range(2)],
            pltpu.SemaphoreType.DMA,
        ],
    )
    return fn(labels2d, featT, cpairT, tailT)


def _loss_body(parts_ref, num_ref, out_ref):
    sumsq = jnp.zeros((16, 1024), jnp.float32)
    for w in range(NW):
        sumsq = sumsq + parts_ref[w]
    dist = jnp.sqrt(sumsq)
    loss = jnp.sum(dist / num_ref[...]) * (1.0 / BATCH)
    out_ref[...] = loss.reshape(1, 1)


@jax.jit
def _tc_stage(parts, num):
    out = pl.pallas_call(
        _loss_body,
        out_shape=jax.ShapeDtypeStruct((1, 1), jnp.float32),
    )(parts.reshape(NW, 16, 1024), num.reshape(16, 1024))
    return out[0, 0]


def kernel(feature, label, centers):
    labels2d = jnp.asarray(label, jnp.int32)
    featT = jnp.transpose(feature)
    centersT = jnp.transpose(centers)
    cpair = centersT.astype(jnp.bfloat16).reshape(FEAT, NPAIR, 2)
    cpairT = jax.lax.bitcast_convert_type(cpair, jnp.float32)
    tailT = jnp.concatenate(
        [cpairT[:, NPAIR - 80:], jnp.zeros((FEAT, 48), jnp.float32)], axis=1)
    parts, num = _sc_stage(labels2d, featT, cpairT, tailT)
    return _tc_stage(parts, num)


# speedup vs baseline: 6.6969x; 6.6969x over previous
import jax
import jax.numpy as jnp
from jax import lax
from jax.experimental import pallas as pl
from jax.experimental.pallas import tpu as pltpu
from jax.experimental.pallas import tpu_sc as plsc

BATCH = 16384
FEAT = 64
NUM_CLASSES = 100000

NC = 2
NS = 16
NW = NC * NS
BPW = BATCH // NW
H0 = 50048
H1 = NUM_CLASSES - H0


def _sc_body(labels_hbm, featT_hbm, centersT_hbm, tailT_hbm,
             parts_hbm, num_hbm,
             dimrow_v, partial_v, labs_v, feat_v, num_v,
             lab_sem, feat_sems, dim_sem):
    c = lax.axis_index("c")
    s = lax.axis_index("s")
    w = c * NS + s

    lab_cp = pltpu.async_copy(labels_hbm, labs_v, lab_sem)
    feat_cps = [
        pltpu.async_copy(featT_hbm.at[c * 32 + p * 16 + s],
                         feat_v[p], feat_sems[p])
        for p in range(2)
    ]
    lab_cp.wait()

    ones = jnp.ones((16,), jnp.float32)
    zeros = jnp.zeros((16,), jnp.float32)

    for h in range(2):
        lo = h * H0
        hsz = H0 if h == 0 else H1

        @plsc.parallel_loop(0, (hsz + 15) // 16, 1, unroll=8)
        def _zero(k):
            dimrow_v[pl.ds(k * 16, 16)] = zeros

        @plsc.parallel_loop(0, BATCH // 16, 1, unroll=8)
        def _count(st):
            lab = labs_v[pl.ds(st * 16, 16)]
            loc = lab - lo
            m = (loc >= 0) & (loc < hsz)
            plsc.addupdate_scatter(dimrow_v, [loc], ones, mask=m)

        @plsc.parallel_loop(0, BPW // 16, 1, unroll=8)
        def _mynum(j):
            lab = labs_v[pl.ds(w * BPW + j * 16, 16)]
            loc = lab - lo
            m = (loc >= 0) & (loc < hsz)
            g = plsc.load_gather(dimrow_v, [loc], mask=m)
            cur = num_v[pl.ds(j * 16, 16)]
            num_v[pl.ds(j * 16, 16)] = jnp.where(m, g, cur)

    ncp = pltpu.async_copy(num_v, num_hbm.at[pl.ds(w * BPW, BPW)], lab_sem)

    for p in range(2):
        feat_cps[p].wait()
        for h in range(2):
            lo = h * H0
            hsz = H0 if h == 0 else H1
            d = c * 32 + p * 16 + s
            if h == 0:
                pltpu.sync_copy(centersT_hbm.at[d, pl.ds(0, H0)],
                                dimrow_v.at[pl.ds(0, H0)])
            else:
                lin = H1 - 32
                pltpu.sync_copy(centersT_hbm.at[d, pl.ds(H0, lin)],
                                dimrow_v.at[pl.ds(0, lin)])
                pltpu.sync_copy(tailT_hbm.at[d],
                                dimrow_v.at[pl.ds(lin, 128)])

            @plsc.parallel_loop(0, BATCH // 16, 1, unroll=8)
            def _sweep(st):
                lab = labs_v[pl.ds(st * 16, 16)]
                loc = lab - lo
                m = (loc >= 0) & (loc < hsz)
                cv = plsc.load_gather(dimrow_v, [loc], mask=m)
                f = feat_v[p][pl.ds(st * 16, 16)]
                d = f - cv
                dd = jnp.where(m, d * d, zeros)
                prow = st >> 6
                pcol = (st & 63) * 16
                if p == 0 and h == 0:
                    partial_v[prow, pl.ds(pcol, 16)] = dd
                else:
                    partial_v[prow, pl.ds(pcol, 16)] = (
                        partial_v[prow, pl.ds(pcol, 16)] + dd)

    ncp.wait()
    pltpu.sync_copy(partial_v, parts_hbm.at[pl.ds(w * 16, 16)])


@jax.jit
def _sc_stage(labels2d, featT, centersT, tailT):
    mesh = plsc.VectorSubcoreMesh(core_axis_name="c", subcore_axis_name="s")
    fn = pl.kernel(
        _sc_body,
        out_type=(
            jax.ShapeDtypeStruct((512, 1024), jnp.float32),
            jax.ShapeDtypeStruct((BATCH,), jnp.float32),
        ),
        mesh=mesh,
        compiler_params=pltpu.CompilerParams(
            needs_layout_passes=False, use_tc_tiling_on_sc=True),
        scratch_types=[
            pltpu.VMEM((H0,), jnp.float32),
            pltpu.VMEM((16, 1024), jnp.float32),
            pltpu.VMEM((BATCH,), jnp.int32),
            [pltpu.VMEM((BATCH,), jnp.float32) for _ in range(2)],
            pltpu.VMEM((BPW,), jnp.float32),
            pltpu.SemaphoreType.DMA,
            [pltpu.SemaphoreType.DMA for _ in range(2)],
            pltpu.SemaphoreType.DMA,
        ],
    )
    return fn(labels2d, featT, centersT, tailT)


def _loss_body(parts_ref, num_ref, out_ref):
    sumsq = jnp.zeros((16, 1024), jnp.float32)
    for w in range(NW):
        sumsq = sumsq + parts_ref[w]
    dist = jnp.sqrt(sumsq)
    loss = jnp.sum(dist / num_ref[...]) * (1.0 / BATCH)
    out_ref[...] = loss.reshape(1, 1)


@jax.jit
def _tc_stage(parts, num):
    out = pl.pallas_call(
        _loss_body,
        out_shape=jax.ShapeDtypeStruct((1, 1), jnp.float32),
    )(parts.reshape(NW, 16, 1024), num.reshape(16, 1024))
    return out[0, 0]


def kernel(feature, label, centers):
    labels2d = jnp.asarray(label, jnp.int32)
    featT = jnp.transpose(feature)
    centersT = jnp.transpose(centers)
    tailT = jnp.concatenate(
        [centersT[:, NUM_CLASSES - 32:], jnp.zeros((FEAT, 96), jnp.float32)],
        axis=1)
    parts, num = _sc_stage(labels2d, featT, centersT, tailT)
    return _tc_stage(parts, num)
